# baseline (speedup 1.0000x reference)
import jax
import jax.numpy as jnp
from jax.experimental import pallas as pl
from jax.experimental.pallas import tpu as pltpu

_BM = 256


def _prologue_body(x_ref, w1t_ref, w2t_ref, b_ref, z_ref, r_ref):
    z_ref[...] = jnp.dot(
        x_ref[...], w2t_ref[...], preferred_element_type=jnp.float32
    )
    r_ref[...] = (
        jnp.dot(x_ref[...], w1t_ref[...], preferred_element_type=jnp.float32)
        + b_ref[...]
    )


def _stream_body(L_ref, z_ref, r_ref, o_ref):
    o_ref[...] = (
        jnp.dot(L_ref[...], z_ref[...], preferred_element_type=jnp.float32)
        + r_ref[...]
    )


@jax.jit
def kernel(L, x, W, b):
    n, d = x.shape
    out = W.shape[0]
    w1t = W[:, :d].T
    w2t = W[:, d:].T
    b2 = b.reshape(1, out)

    z, r = pl.pallas_call(
        _prologue_body,
        out_shape=(
            jax.ShapeDtypeStruct((n, out), jnp.float32),
            jax.ShapeDtypeStruct((n, out), jnp.float32),
        ),
    )(x, w1t, w2t, b2)

    return pl.pallas_call(
        _stream_body,
        grid=(n // _BM,),
        in_specs=[
            pl.BlockSpec((_BM, n), lambda i: (i, 0)),
            pl.BlockSpec((n, out), lambda i: (0, 0)),
            pl.BlockSpec((_BM, out), lambda i: (i, 0)),
        ],
        out_specs=pl.BlockSpec((_BM, out), lambda i: (i, 0)),
        out_shape=jax.ShapeDtypeStruct((n, out), jnp.float32),
        compiler_params=pltpu.CompilerParams(
            dimension_semantics=("parallel",),
        ),
    )(L, z, r)


# speedup vs baseline: 1.0879x; 1.0879x over previous
import jax
import jax.numpy as jnp
from jax.experimental import pallas as pl
from jax.experimental.pallas import tpu as pltpu

_BM = 256
_SPLIT = 4


def _body(*refs):
    L_refs = refs[:_SPLIT]
    x_ref, w1t_ref, w2t_ref, b_ref, o_ref, z_ref = refs[_SPLIT:]
    i = pl.program_id(0)
    kc = z_ref.shape[0] // _SPLIT

    @pl.when(i == 0)
    def _():
        z_ref[...] = jnp.dot(
            x_ref[...], w2t_ref[...], preferred_element_type=jnp.float32
        )

    acc = jnp.dot(
        L_refs[0][...], z_ref[pl.ds(0, kc), :],
        preferred_element_type=jnp.float32,
    )
    for j in range(1, _SPLIT):
        acc += jnp.dot(
            L_refs[j][...], z_ref[pl.ds(j * kc, kc), :],
            preferred_element_type=jnp.float32,
        )
    x_blk = x_ref[pl.ds(i * _BM, _BM), :]
    o_ref[...] = (
        acc
        + jnp.dot(x_blk, w1t_ref[...], preferred_element_type=jnp.float32)
        + b_ref[...]
    )


@jax.jit
def kernel(L, x, W, b):
    n, d = x.shape
    out = W.shape[0]
    w1t = W[:, :d].T
    w2t = W[:, d:].T
    b2 = b.reshape(1, out)
    kc = n // _SPLIT

    def l_spec(j):
        return pl.BlockSpec((_BM, kc), lambda i, j=j: (i, j))

    return pl.pallas_call(
        _body,
        grid=(n // _BM,),
        in_specs=[l_spec(j) for j in range(_SPLIT)]
        + [
            pl.BlockSpec((n, d), lambda i: (0, 0)),
            pl.BlockSpec((d, out), lambda i: (0, 0)),
            pl.BlockSpec((d, out), lambda i: (0, 0)),
            pl.BlockSpec((1, out), lambda i: (0, 0)),
        ],
        out_specs=pl.BlockSpec((_BM, out), lambda i: (i, 0)),
        out_shape=jax.ShapeDtypeStruct((n, out), jnp.float32),
        scratch_shapes=[pltpu.VMEM((n, out), jnp.float32)],
    )(*([L] * _SPLIT), x, w1t, w2t, b2)


# speedup vs baseline: 1.0991x; 1.0103x over previous
import jax
import jax.numpy as jnp
from jax.experimental import pallas as pl
from jax.experimental.pallas import tpu as pltpu

_BM = 256
_SPLIT = 4


def _body(*refs):
    L_refs = refs[:_SPLIT]
    x_ref, w1t_ref, w2t_ref, b_ref, o_ref, z_ref = refs[_SPLIT:]
    i = pl.program_id(0)
    kc = z_ref.shape[0] // _SPLIT

    @pl.when(i == 0)
    def _():
        z_ref[...] = jnp.dot(
            x_ref[...], w2t_ref[...], preferred_element_type=jnp.float32
        )

    acc = jnp.dot(
        L_refs[0][...].astype(jnp.bfloat16),
        z_ref[pl.ds(0, kc), :].astype(jnp.bfloat16),
        preferred_element_type=jnp.float32,
    )
    for j in range(1, _SPLIT):
        acc += jnp.dot(
            L_refs[j][...].astype(jnp.bfloat16),
            z_ref[pl.ds(j * kc, kc), :].astype(jnp.bfloat16),
            preferred_element_type=jnp.float32,
        )
    x_blk = x_ref[pl.ds(i * _BM, _BM), :]
    o_ref[...] = (
        acc
        + jnp.dot(x_blk, w1t_ref[...], preferred_element_type=jnp.float32)
        + b_ref[...]
    )


@jax.jit
def kernel(L, x, W, b):
    n, d = x.shape
    out = W.shape[0]
    w1t = W[:, :d].T
    w2t = W[:, d:].T
    b2 = b.reshape(1, out)
    kc = n // _SPLIT

    def l_spec(j):
        return pl.BlockSpec((_BM, kc), lambda i, j=j: (i, j))

    return pl.pallas_call(
        _body,
        grid=(n // _BM,),
        in_specs=[l_spec(j) for j in range(_SPLIT)]
        + [
            pl.BlockSpec((n, d), lambda i: (0, 0)),
            pl.BlockSpec((d, out), lambda i: (0, 0)),
            pl.BlockSpec((d, out), lambda i: (0, 0)),
            pl.BlockSpec((1, out), lambda i: (0, 0)),
        ],
        out_specs=pl.BlockSpec((_BM, out), lambda i: (i, 0)),
        out_shape=jax.ShapeDtypeStruct((n, out), jnp.float32),
        scratch_shapes=[pltpu.VMEM((n, out), jnp.float32)],
    )(*([L] * _SPLIT), x, w1t, w2t, b2)


# speedup vs baseline: 1.1620x; 1.0572x over previous
import jax
import jax.numpy as jnp
from jax import lax
from jax.experimental import pallas as pl
from jax.experimental.pallas import tpu as pltpu

_BM = 256


def _body(L_ref, xt_ref, w1_ref, w2_ref, b_ref, o_ref, zt_ref, rt_ref):
    i = pl.program_id(0)

    @pl.when(i == 0)
    def _():
        zt_ref[...] = jnp.dot(
            w2_ref[...], xt_ref[...], preferred_element_type=jnp.float32
        )
        rt_ref[...] = (
            jnp.dot(w1_ref[...], xt_ref[...], preferred_element_type=jnp.float32)
            + b_ref[...]
        )

    acc = lax.dot_general(
        zt_ref[...],
        L_ref[...],
        ((( 1,), (1,)), ((), ())),
        preferred_element_type=jnp.float32,
    )
    o_ref[...] = (acc + rt_ref[:, pl.ds(i * _BM, _BM)]).T


@jax.jit
def kernel(L, x, W, b):
    n, d = x.shape
    out = W.shape[0]
    w1 = W[:, :d]
    w2 = W[:, d:]
    xt = x.T
    b2 = b.reshape(out, 1)

    return pl.pallas_call(
        _body,
        grid=(n // _BM,),
        in_specs=[
            pl.BlockSpec((_BM, n), lambda i: (i, 0)),
            pl.BlockSpec((d, n), lambda i: (0, 0)),
            pl.BlockSpec((out, d), lambda i: (0, 0)),
            pl.BlockSpec((out, d), lambda i: (0, 0)),
            pl.BlockSpec((out, 1), lambda i: (0, 0)),
        ],
        out_specs=pl.BlockSpec((_BM, out), lambda i: (i, 0)),
        out_shape=jax.ShapeDtypeStruct((n, out), jnp.float32),
        scratch_shapes=[
            pltpu.VMEM((out, n), jnp.float32),
            pltpu.VMEM((out, n), jnp.float32),
        ],
    )(L, xt, w1, w2, b2)
